# baseline (speedup 1.0000x reference)
import functools

import jax
import jax.numpy as jnp
from jax import lax
from jax.experimental import pallas as pl
from jax.experimental.pallas import tpu as pltpu
from jax.experimental.pallas import tpu_sc as plsc

D = 128
NH = 8
HD = 16
N_SRC = 10000
N_DST = 10000
E = 320000
FF = 2048

NW = 32
CHUNK = 64
SUP = 512
EPW = 10240
E_PAD = NW * EPW
N_ACC = 10112
ZR = N_ACC // 16
AW = D + HD


def _asrc_body(x_ref, w_ref, al32_ref, pm_ref, hbel_ref):
    h = jnp.dot(x_ref[...], w_ref[...], preferred_element_type=jnp.float32)
    hp = jnp.dot(h, pm_ref[...], preferred_element_type=jnp.float32)
    elp = jnp.dot(h, al32_ref[...], preferred_element_type=jnp.float32)
    hbel_ref[...] = jnp.concatenate(
        [hp.astype(jnp.bfloat16), elp.astype(jnp.bfloat16)], axis=-1)


def _gat_pre_src(x, w_gat, al32, permat):
    nb = 5
    rb = N_SRC // nb
    return pl.pallas_call(
        _asrc_body,
        grid=(nb,),
        in_specs=[
            pl.BlockSpec((rb, D), lambda i: (i, 0)),
            pl.BlockSpec((D, D), lambda i: (0, 0)),
            pl.BlockSpec((D, 32), lambda i: (0, 0)),
            pl.BlockSpec((D, D), lambda i: (0, 0)),
        ],
        out_specs=pl.BlockSpec((rb, D + 32), lambda i: (i, 0)),
        out_shape=jax.ShapeDtypeStruct((N_SRC, D + 32), jnp.bfloat16),
    )(x, w_gat, al32, permat)


def _adst_body(x_ref, w_ref, al_ref, ar_ref, h_ref, el_ref, er_ref):
    h = jnp.dot(x_ref[...], w_ref[...], preferred_element_type=jnp.float32)
    h_ref[...] = h
    el_ref[...] = jnp.dot(h, al_ref[...], preferred_element_type=jnp.float32)
    er_ref[...] = jnp.dot(h, ar_ref[...], preferred_element_type=jnp.float32)


def _gat_pre_dst(x, w_gat, almat, armat):
    nb = 5
    rb = N_DST // nb
    return pl.pallas_call(
        _adst_body,
        grid=(nb,),
        in_specs=[
            pl.BlockSpec((rb, D), lambda i: (i, 0)),
            pl.BlockSpec((D, D), lambda i: (0, 0)),
            pl.BlockSpec((D, 16), lambda i: (0, 0)),
            pl.BlockSpec((D, 16), lambda i: (0, 0)),
        ],
        out_specs=[
            pl.BlockSpec((rb, D), lambda i: (i, 0)),
            pl.BlockSpec((rb, 16), lambda i: (i, 0)),
            pl.BlockSpec((rb, 16), lambda i: (i, 0)),
        ],
        out_shape=[
            jax.ShapeDtypeStruct((N_DST, D), jnp.float32),
            jax.ShapeDtypeStruct((N_DST, 16), jnp.float32),
            jax.ShapeDtypeStruct((N_DST, 16), jnp.float32),
        ],
    )(x, w_gat, almat, armat)


def _sc_edge_body(hbel_hbm, er_hbm, src_hbm, dst_hbm,
                  acc0_out, acc1_out,
                  sidx, didx, sdst_v, er_v, hbel_v, msg_v,
                  acc_s,
                  gsem0, gsem1, ssem0, ssem1, isem0, isem1):
    cid = lax.axis_index("c")
    sid = lax.axis_index("s")
    wid = sid * 2 + cid
    gsem = (gsem0, gsem1)
    ssem = (ssem0, ssem1)
    isem = (isem0, isem1)

    zv = jnp.zeros((16,), jnp.float32)
    zb = msg_v.at[0]

    def zrow(r, carry):
        for j in range(AW // 16):
            zb[r, pl.ds(j * 16, 16)] = zv
        return carry

    lax.fori_loop(0, CHUNK, zrow, 0, unroll=4)
    base = sid * ZR
    for j in range(ZR // CHUNK):
        pltpu.sync_copy(zb, acc_s.at[pl.ds(base + j * CHUNK, CHUNK)])
    rem_rows = ZR % CHUNK
    if rem_rows:
        pltpu.sync_copy(zb.at[pl.ds(0, rem_rows)],
                        acc_s.at[pl.ds(base + (ZR // CHUNK) * CHUNK,
                                       rem_rows)])
    plsc.subcore_barrier()

    nchunks = EPW // CHUNK
    cps = SUP // CHUNK
    nsup = EPW // SUP

    def issue_idx(s, B):
        base = wid * EPW + s * SUP
        pltpu.async_copy(src_hbm.at[pl.ds(base, SUP)], sidx.at[B], isem[B])
        pltpu.async_copy(dst_hbm.at[pl.ds(base, SUP)], didx.at[B], isem[B])

    def wait_idx(s, B):
        base = wid * EPW + s * SUP
        pltpu.make_async_copy(src_hbm.at[pl.ds(base, SUP)], sidx.at[B],
                              isem[B]).wait()
        pltpu.make_async_copy(dst_hbm.at[pl.ds(base, SUP)], didx.at[B],
                              isem[B]).wait()

    def issue_gathers(j, B, b):
        src_sl = sidx.at[B, pl.ds(j * CHUNK, CHUNK)]
        dst_sl = didx.at[B, pl.ds(j * CHUNK, CHUNK)]
        pltpu.async_copy(hbel_hbm.at[src_sl], hbel_v.at[b], gsem[b])
        pltpu.async_copy(er_hbm.at[dst_sl], er_v.at[b], gsem[b])

    def wait_gathers(j, B, b):
        src_sl = sidx.at[B, pl.ds(j * CHUNK, CHUNK)]
        dst_sl = didx.at[B, pl.ds(j * CHUNK, CHUNK)]
        pltpu.make_async_copy(hbel_hbm.at[src_sl], hbel_v.at[b],
                              gsem[b]).wait()
        pltpu.make_async_copy(er_hbm.at[dst_sl], er_v.at[b],
                              gsem[b]).wait()

    def wait_scatters(b):
        pltpu.make_async_copy(msg_v.at[b], acc_s.at[sdst_v.at[b]],
                              ssem[b]).wait()

    def compute(b):
        hb, mb = hbel_v.at[b], msg_v.at[b]
        erb = er_v.at[b]

        def edge(k, carry):
            el, _ = plsc.unpack(hb[k, pl.ds(D, 32)],
                                format=plsc.PackFormat.INTERLEAVED)
            e = el + erb[k]
            e = jnp.where(e > 0, e, 0.2 * e)
            w = jnp.exp(e)
            mb[k, pl.ds(D, HD)] = w
            for q in range(NH // 2):
                ha, hb2 = plsc.unpack(hb[k, pl.ds(q * 32, 32)],
                                      format=plsc.PackFormat.INTERLEAVED)
                mb[k, pl.ds((2 * q) * HD, HD)] = ha * w[2 * q]
                mb[k, pl.ds((2 * q + 1) * HD, HD)] = hb2 * w[2 * q + 1]
            return carry

        lax.fori_loop(0, CHUNK, edge, 0, unroll=4)

    issue_idx(0, 0)
    wait_idx(0, 0)
    issue_gathers(0, 0, 0)
    issue_gathers(1, 0, 1)

    def super_pair(sp, carry):
        for S in range(2):
            s = 2 * sp + S

            @pl.when(s + 1 < nsup)
            def _():
                issue_idx(s + 1, 1 - S)

            for j in range(cps):
                b = j % 2
                c = s * cps + j
                wait_gathers(j, S, b)

                @pl.when(c >= 2)
                def _():
                    wait_scatters(b)

                for i in range(CHUNK // 16):
                    sdst_v.at[b][pl.ds(i * 16, 16)] = (
                        didx.at[S][pl.ds(j * CHUNK + i * 16, 16)])
                compute(b)
                pltpu.async_copy(msg_v.at[b], acc_s.at[sdst_v.at[b]],
                                 ssem[b], add=True)

                if j == cps - 2:
                    @pl.when(s + 1 < nsup)
                    def _():
                        wait_idx(s + 1, 1 - S)

                jn = j + 2
                Bn, jn = (S, jn) if jn < cps else (1 - S, jn - cps)

                @pl.when(c + 2 < nchunks)
                def _():
                    issue_gathers(jn, Bn, b)
        return carry

    lax.fori_loop(0, nsup // 2, super_pair, 0)
    for b in range(2):
        wait_scatters(b)
    plsc.subcore_barrier()

    @pl.when(cid == 0)
    def _():
        pltpu.sync_copy(acc_s.at[pl.ds(sid * ZR, ZR)],
                        acc0_out.at[pl.ds(sid * ZR, ZR)])

    @pl.when(cid == 1)
    def _():
        pltpu.sync_copy(acc_s.at[pl.ds(sid * ZR, ZR)],
                        acc1_out.at[pl.ds(sid * ZR, ZR)])


def _sc_edge(hbel, er_dst, src_idx, dst_idx):
    mesh = plsc.VectorSubcoreMesh(core_axis_name="c", subcore_axis_name="s")
    f = pl.kernel(
        _sc_edge_body,
        mesh=mesh,
        compiler_params=pltpu.CompilerParams(use_tc_tiling_on_sc=False,
                                             needs_layout_passes=False),
        out_type=[
            jax.ShapeDtypeStruct((N_ACC, AW), jnp.float32),
            jax.ShapeDtypeStruct((N_ACC, AW), jnp.float32),
        ],
        scratch_types=[
            pltpu.VMEM((2, SUP), jnp.int32),
            pltpu.VMEM((2, SUP), jnp.int32),
            pltpu.VMEM((2, CHUNK), jnp.int32),
            pltpu.VMEM((2, CHUNK, 16), jnp.float32),
            pltpu.VMEM((2, CHUNK, D + 32), jnp.bfloat16),
            pltpu.VMEM((2, CHUNK, AW), jnp.float32),
            pltpu.VMEM_SHARED((N_ACC, AW), jnp.float32),
            pltpu.SemaphoreType.DMA,
            pltpu.SemaphoreType.DMA,
            pltpu.SemaphoreType.DMA,
            pltpu.SemaphoreType.DMA,
            pltpu.SemaphoreType.DMA,
            pltpu.SemaphoreType.DMA,
        ],
    )
    return f(hbel, er_dst, src_idx, dst_idx)


def _ln(x, g, b):
    mu = jnp.mean(x, axis=-1, keepdims=True)
    var = jnp.mean((x - mu) ** 2, axis=-1, keepdims=True)
    return (x - mu) / jnp.sqrt(var + 1e-5) * g + b


def _c1_body(hd_ref, eld_ref, erd_ref, fd_ref,
             wvs_ref, bvs_ref, wos_ref, bos_ref,
             wq_ref, bq_ref, wk_ref, bk_ref, wv_ref, bv_ref,
             g1_ref, b1_ref, erep_ref, ered_ref,
             x1_ref, q_ref, v1_ref, s1_ref, wrep_ref, snum_ref):
    erep = erep_ref[...]
    ered = ered_ref[...]

    eself = eld_ref[...] + erd_ref[...]
    eself = jnp.where(eself > 0, eself, 0.2 * eself)
    wself = jnp.exp(eself)
    wrep = jnp.dot(wself, erep, preferred_element_type=jnp.float32)
    wrep_ref[...] = wrep
    snum_ref[...] = wrep * hd_ref[...]

    t = fd_ref[...]
    sa = jnp.dot(jnp.dot(t, wvs_ref[...], preferred_element_type=jnp.float32),
                 wos_ref[...], preferred_element_type=jnp.float32)
    sa = sa + bvs_ref[...] @ wos_ref[...] + bos_ref[...]
    x = _ln(t + sa, g1_ref[...], b1_ref[...])
    x1_ref[...] = x

    q = jnp.dot(x, wq_ref[...], preferred_element_type=jnp.float32) + bq_ref[...]
    q_ref[...] = q
    k1 = jnp.dot(t, wk_ref[...], preferred_element_type=jnp.float32) + bk_ref[...]
    v1_ref[...] = jnp.dot(t, wv_ref[...],
                          preferred_element_type=jnp.float32) + bv_ref[...]
    s1_ref[...] = jnp.dot(q * k1, ered, preferred_element_type=jnp.float32)


def _c2_body(acc0_ref, acc1_ref, x1_ref, q_ref, v1_ref, s1_ref,
             wrep_ref, snum_ref,
             wk_ref, bk_ref, wv_ref, bv_ref, wo_ref, bo_ref,
             g2_ref, b2_ref, g3_ref, b3_ref,
             w1_ref, bf1_ref, w2_ref, bf2_ref,
             erep_ref, ered_ref,
             out_ref, gat_ref):
    erep = erep_ref[...]
    ered = ered_ref[...]

    den16 = acc0_ref[:, D:] + acc1_ref[:, D:]
    den = jnp.dot(den16, erep, preferred_element_type=jnp.float32) + wrep_ref[...]
    num = acc0_ref[:, :D] + acc1_ref[:, :D] + snum_ref[...]
    gat = num / den
    gat_ref[...] = gat

    q = q_ref[...]
    k2 = jnp.dot(gat, wk_ref[...], preferred_element_type=jnp.float32) + bk_ref[...]
    v2 = jnp.dot(gat, wv_ref[...], preferred_element_type=jnp.float32) + bv_ref[...]
    s2 = jnp.dot(q * k2, ered, preferred_element_type=jnp.float32)
    a1 = 1.0 / (1.0 + jnp.exp(s2 - s1_ref[...]))
    a1r = jnp.dot(a1, erep, preferred_element_type=jnp.float32)
    o = v2 + a1r * (v1_ref[...] - v2)
    ca = jnp.dot(o, wo_ref[...], preferred_element_type=jnp.float32) + bo_ref[...]
    x = _ln(x1_ref[...] + ca, g2_ref[...], b2_ref[...])

    hid = jnp.maximum(
        jnp.dot(x, w1_ref[...], preferred_element_type=jnp.float32)
        + bf1_ref[...], 0.0)
    ff = jnp.dot(hid, w2_ref[...], preferred_element_type=jnp.float32) + bf2_ref[...]
    out_ref[...] = _ln(x + ff, g3_ref[...], b3_ref[...])


def _row_specs(shapes):
    return [pl.BlockSpec(s, lambda i: (i, 0)) for s in shapes]


def _full_specs(arrs):
    return [pl.BlockSpec(a.shape, lambda i: (0, 0)) for a in arrs]


def _decoder_pre(h_dst, el_dst, er_dst, feat_dst, weights):
    nb = 25
    B = N_DST // nb
    return pl.pallas_call(
        _c1_body,
        grid=(nb,),
        in_specs=(_row_specs([(B, D), (B, 16), (B, 16), (B, D)])
                  + _full_specs(weights)),
        out_specs=_row_specs([(B, D), (B, D), (B, D), (B, 16), (B, D),
                              (B, D)]),
        out_shape=[
            jax.ShapeDtypeStruct((N_DST, D), jnp.float32),
            jax.ShapeDtypeStruct((N_DST, D), jnp.float32),
            jax.ShapeDtypeStruct((N_DST, D), jnp.float32),
            jax.ShapeDtypeStruct((N_DST, 16), jnp.float32),
            jax.ShapeDtypeStruct((N_DST, D), jnp.float32),
            jax.ShapeDtypeStruct((N_DST, D), jnp.float32),
        ],
    )(h_dst, el_dst, er_dst, feat_dst, *weights)


def _finalize(acc0, acc1, pre, weights):
    nb = 25
    B = N_DST // nb
    in_specs = (_row_specs([(B, AW), (B, AW), (B, D), (B, D), (B, D),
                            (B, 16), (B, D), (B, D)])
                + _full_specs(weights))
    return pl.pallas_call(
        _c2_body,
        grid=(nb,),
        in_specs=in_specs,
        out_specs=_row_specs([(B, D), (B, D)]),
        out_shape=[
            jax.ShapeDtypeStruct((N_DST, D), jnp.float32),
            jax.ShapeDtypeStruct((N_DST, D), jnp.float32),
        ],
    )(acc0, acc1, *pre, *weights)


def kernel(feat_src, feat_dst, params, edge_index):
    rows = jnp.arange(D)
    colmask = (rows[:, None] // HD) == jnp.arange(16)[None, :]
    almat = jnp.where(colmask, params["attn_l"].reshape(-1)[:, None], 0.0)
    armat = jnp.where(colmask, params["attn_r"].reshape(-1)[:, None], 0.0)
    erep = colmask.astype(jnp.float32).T
    ered = colmask.astype(jnp.float32) * 0.25

    wv = rows // 32
    rem = rows % 32
    half = rem // 16
    ii = rem % 16
    pcol = 32 * wv + 2 * ii + half
    permat = (pcol[:, None] == rows[None, :]).astype(jnp.float32)

    c32 = jnp.arange(32)
    al32 = jnp.where(
        ((rows[:, None] // HD) == (c32[None, :] // 2)) & (c32[None, :] % 2 == 0),
        params["attn_l"].reshape(-1)[:, None], 0.0)

    pad = E_PAD - E
    src_p = jnp.concatenate(
        [edge_index[0].astype(jnp.int32), jnp.zeros((pad,), jnp.int32)])
    dst_p = jnp.concatenate(
        [edge_index[1].astype(jnp.int32),
         jnp.full((pad,), N_DST, jnp.int32)])

    hbel = _gat_pre_src(feat_src, params["W_gat"], al32, permat)
    h_dst, el_dst, er_dst = _gat_pre_dst(feat_dst, params["W_gat"], almat,
                                         armat)

    sa_p, ca_p = params["sa"], params["ca"]

    def r1(v):
        return v.reshape(1, -1)

    w_c1 = [
        sa_p["Wv"], r1(sa_p["bv"]), sa_p["Wo"], r1(sa_p["bo"]),
        ca_p["Wq"], r1(ca_p["bq"]), ca_p["Wk"], r1(ca_p["bk"]),
        ca_p["Wv"], r1(ca_p["bv"]),
        r1(params["ln1_g"]), r1(params["ln1_b"]),
        erep, ered,
    ]
    pre = _decoder_pre(h_dst, el_dst, er_dst, feat_dst, w_c1)

    acc0, acc1 = _sc_edge(hbel, er_dst, src_p, dst_p)

    w_c2 = [
        ca_p["Wk"], r1(ca_p["bk"]), ca_p["Wv"], r1(ca_p["bv"]),
        ca_p["Wo"], r1(ca_p["bo"]),
        r1(params["ln2_g"]), r1(params["ln2_b"]),
        r1(params["ln3_g"]), r1(params["ln3_b"]),
        params["W1"], r1(params["b1"]), params["W2"], r1(params["b2"]),
        erep, ered,
    ]
    out, gat = _finalize(acc0, acc1, pre, w_c2)
    return out, gat


# speedup vs baseline: 1.0086x; 1.0086x over previous
import functools

import jax
import jax.numpy as jnp
from jax import lax
from jax.experimental import pallas as pl
from jax.experimental.pallas import tpu as pltpu
from jax.experimental.pallas import tpu_sc as plsc

D = 128
NH = 8
HD = 16
N_SRC = 10000
N_DST = 10000
E = 320000
FF = 2048

NW = 32
CHUNK = 64
SUP = 512
EPW = 10240
E_PAD = NW * EPW
N_ACC = 10112
ZR = N_ACC // 16
AW = D + HD


def _asrc_body(x_ref, w_ref, al32_ref, pm_ref, hbel_ref):
    h = jnp.dot(x_ref[...], w_ref[...], preferred_element_type=jnp.float32)
    hp = jnp.dot(h, pm_ref[...], preferred_element_type=jnp.float32)
    elp = jnp.dot(h, al32_ref[...], preferred_element_type=jnp.float32)
    hbel_ref[...] = jnp.concatenate(
        [hp.astype(jnp.bfloat16), elp.astype(jnp.bfloat16)], axis=-1)


def _gat_pre_src(x, w_gat, al32, permat):
    nb = 5
    rb = N_SRC // nb
    return pl.pallas_call(
        _asrc_body,
        grid=(nb,),
        in_specs=[
            pl.BlockSpec((rb, D), lambda i: (i, 0)),
            pl.BlockSpec((D, D), lambda i: (0, 0)),
            pl.BlockSpec((D, 32), lambda i: (0, 0)),
            pl.BlockSpec((D, D), lambda i: (0, 0)),
        ],
        out_specs=pl.BlockSpec((rb, D + 32), lambda i: (i, 0)),
        out_shape=jax.ShapeDtypeStruct((N_SRC, D + 32), jnp.bfloat16),
    )(x, w_gat, al32, permat)


def _adst_body(x_ref, w_ref, al_ref, ar_ref, h_ref, el_ref, er_ref):
    h = jnp.dot(x_ref[...], w_ref[...], preferred_element_type=jnp.float32)
    h_ref[...] = h
    el_ref[...] = jnp.dot(h, al_ref[...], preferred_element_type=jnp.float32)
    er_ref[...] = jnp.dot(h, ar_ref[...], preferred_element_type=jnp.float32)


def _gat_pre_dst(x, w_gat, almat, armat):
    nb = 5
    rb = N_DST // nb
    return pl.pallas_call(
        _adst_body,
        grid=(nb,),
        in_specs=[
            pl.BlockSpec((rb, D), lambda i: (i, 0)),
            pl.BlockSpec((D, D), lambda i: (0, 0)),
            pl.BlockSpec((D, 16), lambda i: (0, 0)),
            pl.BlockSpec((D, 16), lambda i: (0, 0)),
        ],
        out_specs=[
            pl.BlockSpec((rb, D), lambda i: (i, 0)),
            pl.BlockSpec((rb, 16), lambda i: (i, 0)),
            pl.BlockSpec((rb, 16), lambda i: (i, 0)),
        ],
        out_shape=[
            jax.ShapeDtypeStruct((N_DST, D), jnp.float32),
            jax.ShapeDtypeStruct((N_DST, 16), jnp.float32),
            jax.ShapeDtypeStruct((N_DST, 16), jnp.float32),
        ],
    )(x, w_gat, almat, armat)


def _sc_edge_body(hbel_hbm, er_hbm, src_hbm, dst_hbm,
                  acc0_out, acc1_out,
                  sidx, didx, sdst_v, er_v, hbel_v, msg_v,
                  acc_s,
                  gsem0, gsem1, ssem0, ssem1, isem0, isem1):
    cid = lax.axis_index("c")
    sid = lax.axis_index("s")
    wid = sid * 2 + cid
    gsem = (gsem0, gsem1)
    ssem = (ssem0, ssem1)
    isem = (isem0, isem1)

    zv = jnp.zeros((16,), jnp.float32)
    zb = msg_v.at[0]

    def zrow(r, carry):
        for j in range(AW // 16):
            zb[r, pl.ds(j * 16, 16)] = zv
        return carry

    lax.fori_loop(0, CHUNK, zrow, 0, unroll=4)
    base = sid * ZR
    for j in range(ZR // CHUNK):
        pltpu.sync_copy(zb, acc_s.at[pl.ds(base + j * CHUNK, CHUNK)])
    rem_rows = ZR % CHUNK
    if rem_rows:
        pltpu.sync_copy(zb.at[pl.ds(0, rem_rows)],
                        acc_s.at[pl.ds(base + (ZR // CHUNK) * CHUNK,
                                       rem_rows)])
    plsc.subcore_barrier()

    nchunks = EPW // CHUNK
    cps = SUP // CHUNK
    nsup = EPW // SUP

    def issue_idx(s, B):
        base = wid * EPW + s * SUP
        pltpu.async_copy(src_hbm.at[pl.ds(base, SUP)], sidx.at[B], isem[B])
        pltpu.async_copy(dst_hbm.at[pl.ds(base, SUP)], didx.at[B], isem[B])

    def wait_idx(s, B):
        base = wid * EPW + s * SUP
        pltpu.make_async_copy(src_hbm.at[pl.ds(base, SUP)], sidx.at[B],
                              isem[B]).wait()
        pltpu.make_async_copy(dst_hbm.at[pl.ds(base, SUP)], didx.at[B],
                              isem[B]).wait()

    def issue_gathers(j, B, b):
        src_sl = sidx.at[B, pl.ds(j * CHUNK, CHUNK)]
        dst_sl = didx.at[B, pl.ds(j * CHUNK, CHUNK)]
        pltpu.async_copy(hbel_hbm.at[src_sl], hbel_v.at[b], gsem[b])
        pltpu.async_copy(er_hbm.at[dst_sl], er_v.at[b], gsem[b])

    def wait_gathers(j, B, b):
        src_sl = sidx.at[B, pl.ds(j * CHUNK, CHUNK)]
        dst_sl = didx.at[B, pl.ds(j * CHUNK, CHUNK)]
        pltpu.make_async_copy(hbel_hbm.at[src_sl], hbel_v.at[b],
                              gsem[b]).wait()
        pltpu.make_async_copy(er_hbm.at[dst_sl], er_v.at[b],
                              gsem[b]).wait()

    def wait_scatters(b):
        pltpu.make_async_copy(msg_v.at[b], acc_s.at[sdst_v.at[b]],
                              ssem[b]).wait()

    def compute(b):
        hb, mb = hbel_v.at[b], msg_v.at[b]
        erb = er_v.at[b]

        def edge(k, carry):
            el, _ = plsc.unpack(hb[k, pl.ds(D, 32)],
                                format=plsc.PackFormat.INTERLEAVED)
            e = el + erb[k]
            e = jnp.where(e > 0, e, 0.2 * e)
            w = jnp.exp(e)
            mb[k, pl.ds(D, HD)] = w
            for q in range(NH // 2):
                ha, hb2 = plsc.unpack(hb[k, pl.ds(q * 32, 32)],
                                      format=plsc.PackFormat.INTERLEAVED)
                mb[k, pl.ds((2 * q) * HD, HD)] = ha * w[2 * q]
                mb[k, pl.ds((2 * q + 1) * HD, HD)] = hb2 * w[2 * q + 1]
            return carry

        lax.fori_loop(0, CHUNK, edge, 0, unroll=2)

    issue_idx(0, 0)
    wait_idx(0, 0)
    issue_gathers(0, 0, 0)
    issue_gathers(1, 0, 1)

    def super_pair(sp, carry):
        for S in range(2):
            s = 2 * sp + S

            @pl.when(s + 1 < nsup)
            def _():
                issue_idx(s + 1, 1 - S)

            for j in range(cps):
                b = j % 2
                c = s * cps + j
                wait_gathers(j, S, b)

                @pl.when(c >= 2)
                def _():
                    wait_scatters(b)

                for i in range(CHUNK // 16):
                    sdst_v.at[b][pl.ds(i * 16, 16)] = (
                        didx.at[S][pl.ds(j * CHUNK + i * 16, 16)])
                compute(b)
                pltpu.async_copy(msg_v.at[b], acc_s.at[sdst_v.at[b]],
                                 ssem[b], add=True)

                if j == cps - 2:
                    @pl.when(s + 1 < nsup)
                    def _():
                        wait_idx(s + 1, 1 - S)

                jn = j + 2
                Bn, jn = (S, jn) if jn < cps else (1 - S, jn - cps)

                @pl.when(c + 2 < nchunks)
                def _():
                    issue_gathers(jn, Bn, b)
        return carry

    lax.fori_loop(0, nsup // 2, super_pair, 0)
    for b in range(2):
        wait_scatters(b)
    plsc.subcore_barrier()

    @pl.when(cid == 0)
    def _():
        pltpu.sync_copy(acc_s.at[pl.ds(sid * ZR, ZR)],
                        acc0_out.at[pl.ds(sid * ZR, ZR)])

    @pl.when(cid == 1)
    def _():
        pltpu.sync_copy(acc_s.at[pl.ds(sid * ZR, ZR)],
                        acc1_out.at[pl.ds(sid * ZR, ZR)])


def _sc_edge(hbel, er_dst, src_idx, dst_idx):
    mesh = plsc.VectorSubcoreMesh(core_axis_name="c", subcore_axis_name="s")
    f = pl.kernel(
        _sc_edge_body,
        mesh=mesh,
        compiler_params=pltpu.CompilerParams(use_tc_tiling_on_sc=False,
                                             needs_layout_passes=False),
        out_type=[
            jax.ShapeDtypeStruct((N_ACC, AW), jnp.float32),
            jax.ShapeDtypeStruct((N_ACC, AW), jnp.float32),
        ],
        scratch_types=[
            pltpu.VMEM((2, SUP), jnp.int32),
            pltpu.VMEM((2, SUP), jnp.int32),
            pltpu.VMEM((2, CHUNK), jnp.int32),
            pltpu.VMEM((2, CHUNK, 16), jnp.float32),
            pltpu.VMEM((2, CHUNK, D + 32), jnp.bfloat16),
            pltpu.VMEM((2, CHUNK, AW), jnp.float32),
            pltpu.VMEM_SHARED((N_ACC, AW), jnp.float32),
            pltpu.SemaphoreType.DMA,
            pltpu.SemaphoreType.DMA,
            pltpu.SemaphoreType.DMA,
            pltpu.SemaphoreType.DMA,
            pltpu.SemaphoreType.DMA,
            pltpu.SemaphoreType.DMA,
        ],
    )
    return f(hbel, er_dst, src_idx, dst_idx)


def _ln(x, g, b):
    mu = jnp.mean(x, axis=-1, keepdims=True)
    var = jnp.mean((x - mu) ** 2, axis=-1, keepdims=True)
    return (x - mu) / jnp.sqrt(var + 1e-5) * g + b


def _c1_body(hd_ref, eld_ref, erd_ref, fd_ref,
             wvs_ref, bvs_ref, wos_ref, bos_ref,
             wq_ref, bq_ref, wk_ref, bk_ref, wv_ref, bv_ref,
             g1_ref, b1_ref, erep_ref, ered_ref,
             x1_ref, q_ref, v1_ref, s1_ref, wrep_ref, snum_ref):
    erep = erep_ref[...]
    ered = ered_ref[...]

    eself = eld_ref[...] + erd_ref[...]
    eself = jnp.where(eself > 0, eself, 0.2 * eself)
    wself = jnp.exp(eself)
    wrep = jnp.dot(wself, erep, preferred_element_type=jnp.float32)
    wrep_ref[...] = wrep
    snum_ref[...] = wrep * hd_ref[...]

    t = fd_ref[...]
    sa = jnp.dot(jnp.dot(t, wvs_ref[...], preferred_element_type=jnp.float32),
                 wos_ref[...], preferred_element_type=jnp.float32)
    sa = sa + bvs_ref[...] @ wos_ref[...] + bos_ref[...]
    x = _ln(t + sa, g1_ref[...], b1_ref[...])
    x1_ref[...] = x

    q = jnp.dot(x, wq_ref[...], preferred_element_type=jnp.float32) + bq_ref[...]
    q_ref[...] = q
    k1 = jnp.dot(t, wk_ref[...], preferred_element_type=jnp.float32) + bk_ref[...]
    v1_ref[...] = jnp.dot(t, wv_ref[...],
                          preferred_element_type=jnp.float32) + bv_ref[...]
    s1_ref[...] = jnp.dot(q * k1, ered, preferred_element_type=jnp.float32)


def _c2_body(acc0_ref, acc1_ref, x1_ref, q_ref, v1_ref, s1_ref,
             wrep_ref, snum_ref,
             wk_ref, bk_ref, wv_ref, bv_ref, wo_ref, bo_ref,
             g2_ref, b2_ref, g3_ref, b3_ref,
             w1_ref, bf1_ref, w2_ref, bf2_ref,
             erep_ref, ered_ref,
             out_ref, gat_ref):
    erep = erep_ref[...]
    ered = ered_ref[...]

    den16 = acc0_ref[:, D:] + acc1_ref[:, D:]
    den = jnp.dot(den16, erep, preferred_element_type=jnp.float32) + wrep_ref[...]
    num = acc0_ref[:, :D] + acc1_ref[:, :D] + snum_ref[...]
    gat = num / den
    gat_ref[...] = gat

    q = q_ref[...]
    k2 = jnp.dot(gat, wk_ref[...], preferred_element_type=jnp.float32) + bk_ref[...]
    v2 = jnp.dot(gat, wv_ref[...], preferred_element_type=jnp.float32) + bv_ref[...]
    s2 = jnp.dot(q * k2, ered, preferred_element_type=jnp.float32)
    a1 = 1.0 / (1.0 + jnp.exp(s2 - s1_ref[...]))
    a1r = jnp.dot(a1, erep, preferred_element_type=jnp.float32)
    o = v2 + a1r * (v1_ref[...] - v2)
    ca = jnp.dot(o, wo_ref[...], preferred_element_type=jnp.float32) + bo_ref[...]
    x = _ln(x1_ref[...] + ca, g2_ref[...], b2_ref[...])

    hid = jnp.maximum(
        jnp.dot(x, w1_ref[...], preferred_element_type=jnp.float32)
        + bf1_ref[...], 0.0)
    ff = jnp.dot(hid, w2_ref[...], preferred_element_type=jnp.float32) + bf2_ref[...]
    out_ref[...] = _ln(x + ff, g3_ref[...], b3_ref[...])


def _row_specs(shapes):
    return [pl.BlockSpec(s, lambda i: (i, 0)) for s in shapes]


def _full_specs(arrs):
    return [pl.BlockSpec(a.shape, lambda i: (0, 0)) for a in arrs]


def _decoder_pre(h_dst, el_dst, er_dst, feat_dst, weights):
    nb = 25
    B = N_DST // nb
    return pl.pallas_call(
        _c1_body,
        grid=(nb,),
        in_specs=(_row_specs([(B, D), (B, 16), (B, 16), (B, D)])
                  + _full_specs(weights)),
        out_specs=_row_specs([(B, D), (B, D), (B, D), (B, 16), (B, D),
                              (B, D)]),
        out_shape=[
            jax.ShapeDtypeStruct((N_DST, D), jnp.float32),
            jax.ShapeDtypeStruct((N_DST, D), jnp.float32),
            jax.ShapeDtypeStruct((N_DST, D), jnp.float32),
            jax.ShapeDtypeStruct((N_DST, 16), jnp.float32),
            jax.ShapeDtypeStruct((N_DST, D), jnp.float32),
            jax.ShapeDtypeStruct((N_DST, D), jnp.float32),
        ],
    )(h_dst, el_dst, er_dst, feat_dst, *weights)


def _finalize(acc0, acc1, pre, weights):
    nb = 25
    B = N_DST // nb
    in_specs = (_row_specs([(B, AW), (B, AW), (B, D), (B, D), (B, D),
                            (B, 16), (B, D), (B, D)])
                + _full_specs(weights))
    return pl.pallas_call(
        _c2_body,
        grid=(nb,),
        in_specs=in_specs,
        out_specs=_row_specs([(B, D), (B, D)]),
        out_shape=[
            jax.ShapeDtypeStruct((N_DST, D), jnp.float32),
            jax.ShapeDtypeStruct((N_DST, D), jnp.float32),
        ],
    )(acc0, acc1, *pre, *weights)


def kernel(feat_src, feat_dst, params, edge_index):
    rows = jnp.arange(D)
    colmask = (rows[:, None] // HD) == jnp.arange(16)[None, :]
    almat = jnp.where(colmask, params["attn_l"].reshape(-1)[:, None], 0.0)
    armat = jnp.where(colmask, params["attn_r"].reshape(-1)[:, None], 0.0)
    erep = colmask.astype(jnp.float32).T
    ered = colmask.astype(jnp.float32) * 0.25

    wv = rows // 32
    rem = rows % 32
    half = rem // 16
    ii = rem % 16
    pcol = 32 * wv + 2 * ii + half
    permat = (pcol[:, None] == rows[None, :]).astype(jnp.float32)

    c32 = jnp.arange(32)
    al32 = jnp.where(
        ((rows[:, None] // HD) == (c32[None, :] // 2)) & (c32[None, :] % 2 == 0),
        params["attn_l"].reshape(-1)[:, None], 0.0)

    pad = E_PAD - E
    src_p = jnp.concatenate(
        [edge_index[0].astype(jnp.int32), jnp.zeros((pad,), jnp.int32)])
    dst_p = jnp.concatenate(
        [edge_index[1].astype(jnp.int32),
         jnp.full((pad,), N_DST, jnp.int32)])

    hbel = _gat_pre_src(feat_src, params["W_gat"], al32, permat)
    h_dst, el_dst, er_dst = _gat_pre_dst(feat_dst, params["W_gat"], almat,
                                         armat)

    sa_p, ca_p = params["sa"], params["ca"]

    def r1(v):
        return v.reshape(1, -1)

    w_c1 = [
        sa_p["Wv"], r1(sa_p["bv"]), sa_p["Wo"], r1(sa_p["bo"]),
        ca_p["Wq"], r1(ca_p["bq"]), ca_p["Wk"], r1(ca_p["bk"]),
        ca_p["Wv"], r1(ca_p["bv"]),
        r1(params["ln1_g"]), r1(params["ln1_b"]),
        erep, ered,
    ]
    pre = _decoder_pre(h_dst, el_dst, er_dst, feat_dst, w_c1)

    acc0, acc1 = _sc_edge(hbel, er_dst, src_p, dst_p)

    w_c2 = [
        ca_p["Wk"], r1(ca_p["bk"]), ca_p["Wv"], r1(ca_p["bv"]),
        ca_p["Wo"], r1(ca_p["bo"]),
        r1(params["ln2_g"]), r1(params["ln2_b"]),
        r1(params["ln3_g"]), r1(params["ln3_b"]),
        params["W1"], r1(params["b1"]), params["W2"], r1(params["b2"]),
        erep, ered,
    ]
    out, gat = _finalize(acc0, acc1, pre, w_c2)
    return out, gat


# speedup vs baseline: 1.0116x; 1.0029x over previous
import functools

import jax
import jax.numpy as jnp
from jax import lax
from jax.experimental import pallas as pl
from jax.experimental.pallas import tpu as pltpu
from jax.experimental.pallas import tpu_sc as plsc

D = 128
NH = 8
HD = 16
N_SRC = 10000
N_DST = 10000
E = 320000
FF = 2048

NW = 32
CHUNK = 64
SUP = 512
EPW = 10240
E_PAD = NW * EPW
N_ACC = 10112
ZR = N_ACC // 16
AW = D + HD


def _asrc_body(x_ref, w_ref, al32_ref, pm_ref, hbel_ref):
    h = jnp.dot(x_ref[...], w_ref[...], preferred_element_type=jnp.float32)
    hp = jnp.dot(h, pm_ref[...], preferred_element_type=jnp.float32)
    elp = jnp.dot(h, al32_ref[...], preferred_element_type=jnp.float32)
    hbel_ref[...] = jnp.concatenate(
        [hp.astype(jnp.bfloat16), elp.astype(jnp.bfloat16)], axis=-1)


def _gat_pre_src(x, w_gat, al32, permat):
    nb = 5
    rb = N_SRC // nb
    return pl.pallas_call(
        _asrc_body,
        grid=(nb,),
        in_specs=[
            pl.BlockSpec((rb, D), lambda i: (i, 0)),
            pl.BlockSpec((D, D), lambda i: (0, 0)),
            pl.BlockSpec((D, 32), lambda i: (0, 0)),
            pl.BlockSpec((D, D), lambda i: (0, 0)),
        ],
        out_specs=pl.BlockSpec((rb, D + 32), lambda i: (i, 0)),
        out_shape=jax.ShapeDtypeStruct((N_SRC, D + 32), jnp.bfloat16),
    )(x, w_gat, al32, permat)


def _adst_body(x_ref, w_ref, al_ref, ar_ref, h_ref, el_ref, er_ref):
    h = jnp.dot(x_ref[...], w_ref[...], preferred_element_type=jnp.float32)
    h_ref[...] = h
    el_ref[...] = jnp.dot(h, al_ref[...], preferred_element_type=jnp.float32)
    er_ref[...] = jnp.dot(h, ar_ref[...], preferred_element_type=jnp.float32)


def _gat_pre_dst(x, w_gat, almat, armat):
    nb = 5
    rb = N_DST // nb
    return pl.pallas_call(
        _adst_body,
        grid=(nb,),
        in_specs=[
            pl.BlockSpec((rb, D), lambda i: (i, 0)),
            pl.BlockSpec((D, D), lambda i: (0, 0)),
            pl.BlockSpec((D, 16), lambda i: (0, 0)),
            pl.BlockSpec((D, 16), lambda i: (0, 0)),
        ],
        out_specs=[
            pl.BlockSpec((rb, D), lambda i: (i, 0)),
            pl.BlockSpec((rb, 16), lambda i: (i, 0)),
            pl.BlockSpec((rb, 16), lambda i: (i, 0)),
        ],
        out_shape=[
            jax.ShapeDtypeStruct((N_DST, D), jnp.float32),
            jax.ShapeDtypeStruct((N_DST, 16), jnp.float32),
            jax.ShapeDtypeStruct((N_DST, 16), jnp.float32),
        ],
    )(x, w_gat, almat, armat)


def _sc_edge_body(hbel_hbm, er_hbm, src_hbm, dst_hbm,
                  acc0_out, acc1_out,
                  sidx, didx, sdst_v, er_v, hbel_v, msg_v,
                  acc_s,
                  gsem0, gsem1, ssem0, ssem1, isem0, isem1):
    cid = lax.axis_index("c")
    sid = lax.axis_index("s")
    wid = sid * 2 + cid
    gsem = (gsem0, gsem1)
    ssem = (ssem0, ssem1)
    isem = (isem0, isem1)

    nchunks = EPW // CHUNK
    cps = SUP // CHUNK
    nsup = EPW // SUP

    def issue_idx(s, B):
        base = wid * EPW + s * SUP
        pltpu.async_copy(src_hbm.at[pl.ds(base, SUP)], sidx.at[B], isem[B])
        pltpu.async_copy(dst_hbm.at[pl.ds(base, SUP)], didx.at[B], isem[B])

    def wait_idx(s, B):
        base = wid * EPW + s * SUP
        pltpu.make_async_copy(src_hbm.at[pl.ds(base, SUP)], sidx.at[B],
                              isem[B]).wait()
        pltpu.make_async_copy(dst_hbm.at[pl.ds(base, SUP)], didx.at[B],
                              isem[B]).wait()

    def issue_gathers(j, B, b):
        src_sl = sidx.at[B, pl.ds(j * CHUNK, CHUNK)]
        dst_sl = didx.at[B, pl.ds(j * CHUNK, CHUNK)]
        pltpu.async_copy(hbel_hbm.at[src_sl], hbel_v.at[b], gsem[b])
        pltpu.async_copy(er_hbm.at[dst_sl], er_v.at[b], gsem[b])

    def wait_gathers(j, B, b):
        src_sl = sidx.at[B, pl.ds(j * CHUNK, CHUNK)]
        dst_sl = didx.at[B, pl.ds(j * CHUNK, CHUNK)]
        pltpu.make_async_copy(hbel_hbm.at[src_sl], hbel_v.at[b],
                              gsem[b]).wait()
        pltpu.make_async_copy(er_hbm.at[dst_sl], er_v.at[b],
                              gsem[b]).wait()

    def wait_scatters(b):
        pltpu.make_async_copy(msg_v.at[b], acc_s.at[sdst_v.at[b]],
                              ssem[b]).wait()

    def compute(b):
        hb, mb = hbel_v.at[b], msg_v.at[b]
        erb = er_v.at[b]

        def edge(k, carry):
            el, _ = plsc.unpack(hb[k, pl.ds(D, 32)],
                                format=plsc.PackFormat.INTERLEAVED)
            e = el + erb[k]
            e = jnp.where(e > 0, e, 0.2 * e)
            w = jnp.exp(e)
            mb[k, pl.ds(D, HD)] = w
            for q in range(NH // 2):
                ha, hb2 = plsc.unpack(hb[k, pl.ds(q * 32, 32)],
                                      format=plsc.PackFormat.INTERLEAVED)
                mb[k, pl.ds((2 * q) * HD, HD)] = ha * w[2 * q]
                mb[k, pl.ds((2 * q + 1) * HD, HD)] = hb2 * w[2 * q + 1]
            return carry

        lax.fori_loop(0, CHUNK, edge, 0, unroll=2)

    issue_idx(0, 0)
    zv = jnp.zeros((16,), jnp.float32)
    zb = msg_v.at[0]

    def zrow(r, carry):
        for j in range(AW // 16):
            zb[r, pl.ds(j * 16, 16)] = zv
        return carry

    lax.fori_loop(0, CHUNK, zrow, 0, unroll=4)
    wait_idx(0, 0)
    issue_gathers(0, 0, 0)
    issue_gathers(1, 0, 1)
    zbase = sid * ZR
    for j in range(ZR // CHUNK):
        pltpu.sync_copy(zb, acc_s.at[pl.ds(zbase + j * CHUNK, CHUNK)])
    if ZR % CHUNK:
        pltpu.sync_copy(zb.at[pl.ds(0, ZR % CHUNK)],
                        acc_s.at[pl.ds(zbase + (ZR // CHUNK) * CHUNK,
                                       ZR % CHUNK)])
    plsc.subcore_barrier()

    def super_pair(sp, carry):
        for S in range(2):
            s = 2 * sp + S

            @pl.when(s + 1 < nsup)
            def _():
                issue_idx(s + 1, 1 - S)

            for j in range(cps):
                b = j % 2
                c = s * cps + j
                wait_gathers(j, S, b)

                @pl.when(c >= 2)
                def _():
                    wait_scatters(b)

                for i in range(CHUNK // 16):
                    sdst_v.at[b][pl.ds(i * 16, 16)] = (
                        didx.at[S][pl.ds(j * CHUNK + i * 16, 16)])
                compute(b)
                pltpu.async_copy(msg_v.at[b], acc_s.at[sdst_v.at[b]],
                                 ssem[b], add=True)

                if j == cps - 2:
                    @pl.when(s + 1 < nsup)
                    def _():
                        wait_idx(s + 1, 1 - S)

                jn = j + 2
                Bn, jn = (S, jn) if jn < cps else (1 - S, jn - cps)

                @pl.when(c + 2 < nchunks)
                def _():
                    issue_gathers(jn, Bn, b)
        return carry

    lax.fori_loop(0, nsup // 2, super_pair, 0)
    for b in range(2):
        wait_scatters(b)
    plsc.subcore_barrier()

    @pl.when(cid == 0)
    def _():
        pltpu.sync_copy(acc_s.at[pl.ds(sid * ZR, ZR)],
                        acc0_out.at[pl.ds(sid * ZR, ZR)])

    @pl.when(cid == 1)
    def _():
        pltpu.sync_copy(acc_s.at[pl.ds(sid * ZR, ZR)],
                        acc1_out.at[pl.ds(sid * ZR, ZR)])


def _sc_edge(hbel, er_dst, src_idx, dst_idx):
    mesh = plsc.VectorSubcoreMesh(core_axis_name="c", subcore_axis_name="s")
    f = pl.kernel(
        _sc_edge_body,
        mesh=mesh,
        compiler_params=pltpu.CompilerParams(use_tc_tiling_on_sc=False,
                                             needs_layout_passes=False),
        out_type=[
            jax.ShapeDtypeStruct((N_ACC, AW), jnp.float32),
            jax.ShapeDtypeStruct((N_ACC, AW), jnp.float32),
        ],
        scratch_types=[
            pltpu.VMEM((2, SUP), jnp.int32),
            pltpu.VMEM((2, SUP), jnp.int32),
            pltpu.VMEM((2, CHUNK), jnp.int32),
            pltpu.VMEM((2, CHUNK, 16), jnp.float32),
            pltpu.VMEM((2, CHUNK, D + 32), jnp.bfloat16),
            pltpu.VMEM((2, CHUNK, AW), jnp.float32),
            pltpu.VMEM_SHARED((N_ACC, AW), jnp.float32),
            pltpu.SemaphoreType.DMA,
            pltpu.SemaphoreType.DMA,
            pltpu.SemaphoreType.DMA,
            pltpu.SemaphoreType.DMA,
            pltpu.SemaphoreType.DMA,
            pltpu.SemaphoreType.DMA,
        ],
    )
    return f(hbel, er_dst, src_idx, dst_idx)


def _ln(x, g, b):
    mu = jnp.mean(x, axis=-1, keepdims=True)
    var = jnp.mean((x - mu) ** 2, axis=-1, keepdims=True)
    return (x - mu) / jnp.sqrt(var + 1e-5) * g + b


def _c1_body(hd_ref, eld_ref, erd_ref, fd_ref,
             wvs_ref, bvs_ref, wos_ref, bos_ref,
             wq_ref, bq_ref, wk_ref, bk_ref, wv_ref, bv_ref,
             g1_ref, b1_ref, erep_ref, ered_ref,
             x1_ref, q_ref, v1_ref, s1_ref, wrep_ref, snum_ref):
    erep = erep_ref[...]
    ered = ered_ref[...]

    eself = eld_ref[...] + erd_ref[...]
    eself = jnp.where(eself > 0, eself, 0.2 * eself)
    wself = jnp.exp(eself)
    wrep = jnp.dot(wself, erep, preferred_element_type=jnp.float32)
    wrep_ref[...] = wrep
    snum_ref[...] = wrep * hd_ref[...]

    t = fd_ref[...]
    sa = jnp.dot(jnp.dot(t, wvs_ref[...], preferred_element_type=jnp.float32),
                 wos_ref[...], preferred_element_type=jnp.float32)
    sa = sa + bvs_ref[...] @ wos_ref[...] + bos_ref[...]
    x = _ln(t + sa, g1_ref[...], b1_ref[...])
    x1_ref[...] = x

    q = jnp.dot(x, wq_ref[...], preferred_element_type=jnp.float32) + bq_ref[...]
    q_ref[...] = q
    k1 = jnp.dot(t, wk_ref[...], preferred_element_type=jnp.float32) + bk_ref[...]
    v1_ref[...] = jnp.dot(t, wv_ref[...],
                          preferred_element_type=jnp.float32) + bv_ref[...]
    s1_ref[...] = jnp.dot(q * k1, ered, preferred_element_type=jnp.float32)


def _c2_body(acc0_ref, acc1_ref, x1_ref, q_ref, v1_ref, s1_ref,
             wrep_ref, snum_ref,
             wk_ref, bk_ref, wv_ref, bv_ref, wo_ref, bo_ref,
             g2_ref, b2_ref, g3_ref, b3_ref,
             w1_ref, bf1_ref, w2_ref, bf2_ref,
             erep_ref, ered_ref,
             out_ref, gat_ref):
    erep = erep_ref[...]
    ered = ered_ref[...]

    den16 = acc0_ref[:, D:] + acc1_ref[:, D:]
    den = jnp.dot(den16, erep, preferred_element_type=jnp.float32) + wrep_ref[...]
    num = acc0_ref[:, :D] + acc1_ref[:, :D] + snum_ref[...]
    gat = num / den
    gat_ref[...] = gat

    q = q_ref[...]
    k2 = jnp.dot(gat, wk_ref[...], preferred_element_type=jnp.float32) + bk_ref[...]
    v2 = jnp.dot(gat, wv_ref[...], preferred_element_type=jnp.float32) + bv_ref[...]
    s2 = jnp.dot(q * k2, ered, preferred_element_type=jnp.float32)
    a1 = 1.0 / (1.0 + jnp.exp(s2 - s1_ref[...]))
    a1r = jnp.dot(a1, erep, preferred_element_type=jnp.float32)
    o = v2 + a1r * (v1_ref[...] - v2)
    ca = jnp.dot(o, wo_ref[...], preferred_element_type=jnp.float32) + bo_ref[...]
    x = _ln(x1_ref[...] + ca, g2_ref[...], b2_ref[...])

    hid = jnp.maximum(
        jnp.dot(x, w1_ref[...], preferred_element_type=jnp.float32)
        + bf1_ref[...], 0.0)
    ff = jnp.dot(hid, w2_ref[...], preferred_element_type=jnp.float32) + bf2_ref[...]
    out_ref[...] = _ln(x + ff, g3_ref[...], b3_ref[...])


def _row_specs(shapes):
    return [pl.BlockSpec(s, lambda i: (i, 0)) for s in shapes]


def _full_specs(arrs):
    return [pl.BlockSpec(a.shape, lambda i: (0, 0)) for a in arrs]


def _decoder_pre(h_dst, el_dst, er_dst, feat_dst, weights):
    nb = 25
    B = N_DST // nb
    return pl.pallas_call(
        _c1_body,
        grid=(nb,),
        in_specs=(_row_specs([(B, D), (B, 16), (B, 16), (B, D)])
                  + _full_specs(weights)),
        out_specs=_row_specs([(B, D), (B, D), (B, D), (B, 16), (B, D),
                              (B, D)]),
        out_shape=[
            jax.ShapeDtypeStruct((N_DST, D), jnp.float32),
            jax.ShapeDtypeStruct((N_DST, D), jnp.float32),
            jax.ShapeDtypeStruct((N_DST, D), jnp.float32),
            jax.ShapeDtypeStruct((N_DST, 16), jnp.float32),
            jax.ShapeDtypeStruct((N_DST, D), jnp.float32),
            jax.ShapeDtypeStruct((N_DST, D), jnp.float32),
        ],
    )(h_dst, el_dst, er_dst, feat_dst, *weights)


def _finalize(acc0, acc1, pre, weights):
    nb = 25
    B = N_DST // nb
    in_specs = (_row_specs([(B, AW), (B, AW), (B, D), (B, D), (B, D),
                            (B, 16), (B, D), (B, D)])
                + _full_specs(weights))
    return pl.pallas_call(
        _c2_body,
        grid=(nb,),
        in_specs=in_specs,
        out_specs=_row_specs([(B, D), (B, D)]),
        out_shape=[
            jax.ShapeDtypeStruct((N_DST, D), jnp.float32),
            jax.ShapeDtypeStruct((N_DST, D), jnp.float32),
        ],
    )(acc0, acc1, *pre, *weights)


def kernel(feat_src, feat_dst, params, edge_index):
    rows = jnp.arange(D)
    colmask = (rows[:, None] // HD) == jnp.arange(16)[None, :]
    almat = jnp.where(colmask, params["attn_l"].reshape(-1)[:, None], 0.0)
    armat = jnp.where(colmask, params["attn_r"].reshape(-1)[:, None], 0.0)
    erep = colmask.astype(jnp.float32).T
    ered = colmask.astype(jnp.float32) * 0.25

    wv = rows // 32
    rem = rows % 32
    half = rem // 16
    ii = rem % 16
    pcol = 32 * wv + 2 * ii + half
    permat = (pcol[:, None] == rows[None, :]).astype(jnp.float32)

    c32 = jnp.arange(32)
    al32 = jnp.where(
        ((rows[:, None] // HD) == (c32[None, :] // 2)) & (c32[None, :] % 2 == 0),
        params["attn_l"].reshape(-1)[:, None], 0.0)

    pad = E_PAD - E
    src_p = jnp.concatenate(
        [edge_index[0].astype(jnp.int32), jnp.zeros((pad,), jnp.int32)])
    dst_p = jnp.concatenate(
        [edge_index[1].astype(jnp.int32),
         jnp.full((pad,), N_DST, jnp.int32)])

    hbel = _gat_pre_src(feat_src, params["W_gat"], al32, permat)
    h_dst, el_dst, er_dst = _gat_pre_dst(feat_dst, params["W_gat"], almat,
                                         armat)

    sa_p, ca_p = params["sa"], params["ca"]

    def r1(v):
        return v.reshape(1, -1)

    w_c1 = [
        sa_p["Wv"], r1(sa_p["bv"]), sa_p["Wo"], r1(sa_p["bo"]),
        ca_p["Wq"], r1(ca_p["bq"]), ca_p["Wk"], r1(ca_p["bk"]),
        ca_p["Wv"], r1(ca_p["bv"]),
        r1(params["ln1_g"]), r1(params["ln1_b"]),
        erep, ered,
    ]
    pre = _decoder_pre(h_dst, el_dst, er_dst, feat_dst, w_c1)

    acc0, acc1 = _sc_edge(hbel, er_dst, src_p, dst_p)

    w_c2 = [
        ca_p["Wk"], r1(ca_p["bk"]), ca_p["Wv"], r1(ca_p["bv"]),
        ca_p["Wo"], r1(ca_p["bo"]),
        r1(params["ln2_g"]), r1(params["ln2_b"]),
        r1(params["ln3_g"]), r1(params["ln3_b"]),
        params["W1"], r1(params["b1"]), params["W2"], r1(params["b2"]),
        erep, ered,
    ]
    out, gat = _finalize(acc0, acc1, pre, w_c2)
    return out, gat


# speedup vs baseline: 1.0543x; 1.0422x over previous
import functools

import jax
import jax.numpy as jnp
from jax import lax
from jax.experimental import pallas as pl
from jax.experimental.pallas import tpu as pltpu
from jax.experimental.pallas import tpu_sc as plsc

D = 128
NH = 8
HD = 16
N_SRC = 10000
N_DST = 10000
E = 320000
FF = 2048

NW = 32
CHUNK = 64
SUP = 512
EPW = 10240
E_PAD = NW * EPW
N_ACC = 10112
ZR = N_ACC // 16
AW = D + HD


def _pre_body(xs_ref, xd_ref, w_ref, al32_ref, pm_ref, al_ref, ar_ref,
              wvs_ref, bvs_ref, wos_ref, bos_ref,
              wq_ref, bq_ref, wk_ref, bk_ref, wv_ref, bv_ref,
              g1_ref, b1_ref, erep_ref, ered_ref,
              hbel_ref, er_ref, x1_ref, q_ref, v1_ref, s1_ref,
              wrep_ref, snum_ref):
    hs = jnp.dot(xs_ref[...], w_ref[...], preferred_element_type=jnp.float32)
    hp = jnp.dot(hs, pm_ref[...], preferred_element_type=jnp.float32)
    elp = jnp.dot(hs, al32_ref[...], preferred_element_type=jnp.float32)
    hbel_ref[...] = jnp.concatenate(
        [hp.astype(jnp.bfloat16), elp.astype(jnp.bfloat16)], axis=-1)

    hd = jnp.dot(xd_ref[...], w_ref[...], preferred_element_type=jnp.float32)
    el = jnp.dot(hd, al_ref[...], preferred_element_type=jnp.float32)
    er = jnp.dot(hd, ar_ref[...], preferred_element_type=jnp.float32)
    er_ref[...] = er

    erep = erep_ref[...]
    ered = ered_ref[...]
    eself = el + er
    eself = jnp.where(eself > 0, eself, 0.2 * eself)
    wself = jnp.exp(eself)
    wrep = jnp.dot(wself, erep, preferred_element_type=jnp.float32)
    wrep_ref[...] = wrep
    snum_ref[...] = wrep * hd

    t = xd_ref[...]
    sa = jnp.dot(jnp.dot(t, wvs_ref[...], preferred_element_type=jnp.float32),
                 wos_ref[...], preferred_element_type=jnp.float32)
    sa = sa + bvs_ref[...] @ wos_ref[...] + bos_ref[...]
    x = _ln(t + sa, g1_ref[...], b1_ref[...])
    x1_ref[...] = x

    q = jnp.dot(x, wq_ref[...], preferred_element_type=jnp.float32) + bq_ref[...]
    q_ref[...] = q
    k1 = jnp.dot(t, wk_ref[...], preferred_element_type=jnp.float32) + bk_ref[...]
    v1_ref[...] = jnp.dot(t, wv_ref[...],
                          preferred_element_type=jnp.float32) + bv_ref[...]
    s1_ref[...] = jnp.dot(q * k1, ered, preferred_element_type=jnp.float32)


def _gat_pre(feat_src, feat_dst, w_gat, al32, permat, almat, armat, w_c1):
    nb = 5
    rb = N_SRC // nb
    mats = [w_gat, al32, permat, almat, armat] + list(w_c1)
    return pl.pallas_call(
        _pre_body,
        grid=(nb,),
        in_specs=([pl.BlockSpec((rb, D), lambda i: (i, 0)),
                   pl.BlockSpec((rb, D), lambda i: (i, 0))]
                  + _full_specs(mats)),
        out_specs=_row_specs([(rb, D + 32), (rb, 16), (rb, D), (rb, D),
                              (rb, D), (rb, 16), (rb, D), (rb, D)]),
        out_shape=[
            jax.ShapeDtypeStruct((N_SRC, D + 32), jnp.bfloat16),
            jax.ShapeDtypeStruct((N_DST, 16), jnp.float32),
            jax.ShapeDtypeStruct((N_DST, D), jnp.float32),
            jax.ShapeDtypeStruct((N_DST, D), jnp.float32),
            jax.ShapeDtypeStruct((N_DST, D), jnp.float32),
            jax.ShapeDtypeStruct((N_DST, 16), jnp.float32),
            jax.ShapeDtypeStruct((N_DST, D), jnp.float32),
            jax.ShapeDtypeStruct((N_DST, D), jnp.float32),
        ],
    )(feat_src, feat_dst, *mats)


def _sc_edge_body(hbel_hbm, er_hbm, src_hbm, dst_hbm,
                  acc0_out, acc1_out,
                  sidx, didx, sdst_v, er_v, hbel_v, msg_v,
                  acc_s,
                  gsem0, gsem1, ssem0, ssem1, isem0, isem1):
    cid = lax.axis_index("c")
    sid = lax.axis_index("s")
    wid = sid * 2 + cid
    gsem = (gsem0, gsem1)
    ssem = (ssem0, ssem1)
    isem = (isem0, isem1)

    nchunks = EPW // CHUNK
    cps = SUP // CHUNK
    nsup = EPW // SUP

    def issue_idx(s, B):
        base = wid * EPW + s * SUP
        pltpu.async_copy(src_hbm.at[pl.ds(base, SUP)], sidx.at[B], isem[B])
        pltpu.async_copy(dst_hbm.at[pl.ds(base, SUP)], didx.at[B], isem[B])

    def wait_idx(s, B):
        base = wid * EPW + s * SUP
        pltpu.make_async_copy(src_hbm.at[pl.ds(base, SUP)], sidx.at[B],
                              isem[B]).wait()
        pltpu.make_async_copy(dst_hbm.at[pl.ds(base, SUP)], didx.at[B],
                              isem[B]).wait()

    def issue_gathers(j, B, b):
        src_sl = sidx.at[B, pl.ds(j * CHUNK, CHUNK)]
        dst_sl = didx.at[B, pl.ds(j * CHUNK, CHUNK)]
        pltpu.async_copy(hbel_hbm.at[src_sl], hbel_v.at[b], gsem[b])
        pltpu.async_copy(er_hbm.at[dst_sl], er_v.at[b], gsem[b])

    def wait_gathers(j, B, b):
        src_sl = sidx.at[B, pl.ds(j * CHUNK, CHUNK)]
        dst_sl = didx.at[B, pl.ds(j * CHUNK, CHUNK)]
        pltpu.make_async_copy(hbel_hbm.at[src_sl], hbel_v.at[b],
                              gsem[b]).wait()
        pltpu.make_async_copy(er_hbm.at[dst_sl], er_v.at[b],
                              gsem[b]).wait()

    def wait_scatters(b):
        pltpu.make_async_copy(msg_v.at[b], acc_s.at[sdst_v.at[b]],
                              ssem[b]).wait()

    def compute(b):
        hb, mb = hbel_v.at[b], msg_v.at[b]
        erb = er_v.at[b]

        def edge(k, carry):
            el, _ = plsc.unpack(hb[k, pl.ds(D, 32)],
                                format=plsc.PackFormat.INTERLEAVED)
            e = el + erb[k]
            e = jnp.where(e > 0, e, 0.2 * e)
            w = jnp.exp(e)
            mb[k, pl.ds(D, HD)] = w
            for q in range(NH // 2):
                ha, hb2 = plsc.unpack(hb[k, pl.ds(q * 32, 32)],
                                      format=plsc.PackFormat.INTERLEAVED)
                mb[k, pl.ds((2 * q) * HD, HD)] = ha * w[2 * q]
                mb[k, pl.ds((2 * q + 1) * HD, HD)] = hb2 * w[2 * q + 1]
            return carry

        lax.fori_loop(0, CHUNK, edge, 0, unroll=2)

    issue_idx(0, 0)
    zv = jnp.zeros((16,), jnp.float32)
    zb = msg_v.at[0]

    def zrow(r, carry):
        for j in range(AW // 16):
            zb[r, pl.ds(j * 16, 16)] = zv
        return carry

    lax.fori_loop(0, CHUNK, zrow, 0, unroll=4)
    wait_idx(0, 0)
    issue_gathers(0, 0, 0)
    issue_gathers(1, 0, 1)
    zbase = sid * ZR
    for j in range(ZR // CHUNK):
        pltpu.sync_copy(zb, acc_s.at[pl.ds(zbase + j * CHUNK, CHUNK)])
    if ZR % CHUNK:
        pltpu.sync_copy(zb.at[pl.ds(0, ZR % CHUNK)],
                        acc_s.at[pl.ds(zbase + (ZR // CHUNK) * CHUNK,
                                       ZR % CHUNK)])
    plsc.subcore_barrier()

    def super_pair(sp, carry):
        for S in range(2):
            s = 2 * sp + S

            @pl.when(s + 1 < nsup)
            def _():
                issue_idx(s + 1, 1 - S)

            for j in range(cps):
                b = j % 2
                c = s * cps + j
                wait_gathers(j, S, b)

                @pl.when(c >= 2)
                def _():
                    wait_scatters(b)

                for i in range(CHUNK // 16):
                    sdst_v.at[b][pl.ds(i * 16, 16)] = (
                        didx.at[S][pl.ds(j * CHUNK + i * 16, 16)])
                compute(b)
                pltpu.async_copy(msg_v.at[b], acc_s.at[sdst_v.at[b]],
                                 ssem[b], add=True)

                if j == cps - 2:
                    @pl.when(s + 1 < nsup)
                    def _():
                        wait_idx(s + 1, 1 - S)

                jn = j + 2
                Bn, jn = (S, jn) if jn < cps else (1 - S, jn - cps)

                @pl.when(c + 2 < nchunks)
                def _():
                    issue_gathers(jn, Bn, b)
        return carry

    lax.fori_loop(0, nsup // 2, super_pair, 0)
    for b in range(2):
        wait_scatters(b)
    plsc.subcore_barrier()

    @pl.when(cid == 0)
    def _():
        pltpu.sync_copy(acc_s.at[pl.ds(sid * ZR, ZR)],
                        acc0_out.at[pl.ds(sid * ZR, ZR)])

    @pl.when(cid == 1)
    def _():
        pltpu.sync_copy(acc_s.at[pl.ds(sid * ZR, ZR)],
                        acc1_out.at[pl.ds(sid * ZR, ZR)])


def _sc_edge(hbel, er_dst, src_idx, dst_idx):
    mesh = plsc.VectorSubcoreMesh(core_axis_name="c", subcore_axis_name="s")
    f = pl.kernel(
        _sc_edge_body,
        mesh=mesh,
        compiler_params=pltpu.CompilerParams(use_tc_tiling_on_sc=False,
                                             needs_layout_passes=False),
        out_type=[
            jax.ShapeDtypeStruct((N_ACC, AW), jnp.float32),
            jax.ShapeDtypeStruct((N_ACC, AW), jnp.float32),
        ],
        scratch_types=[
            pltpu.VMEM((2, SUP), jnp.int32),
            pltpu.VMEM((2, SUP), jnp.int32),
            pltpu.VMEM((2, CHUNK), jnp.int32),
            pltpu.VMEM((2, CHUNK, 16), jnp.float32),
            pltpu.VMEM((2, CHUNK, D + 32), jnp.bfloat16),
            pltpu.VMEM((2, CHUNK, AW), jnp.float32),
            pltpu.VMEM_SHARED((N_ACC, AW), jnp.float32),
            pltpu.SemaphoreType.DMA,
            pltpu.SemaphoreType.DMA,
            pltpu.SemaphoreType.DMA,
            pltpu.SemaphoreType.DMA,
            pltpu.SemaphoreType.DMA,
            pltpu.SemaphoreType.DMA,
        ],
    )
    return f(hbel, er_dst, src_idx, dst_idx)


def _ln(x, g, b):
    mu = jnp.mean(x, axis=-1, keepdims=True)
    var = jnp.mean((x - mu) ** 2, axis=-1, keepdims=True)
    return (x - mu) / jnp.sqrt(var + 1e-5) * g + b


def _c2_body(acc0_ref, acc1_ref, x1_ref, q_ref, v1_ref, s1_ref,
             wrep_ref, snum_ref,
             wk_ref, bk_ref, wv_ref, bv_ref, wo_ref, bo_ref,
             g2_ref, b2_ref, g3_ref, b3_ref,
             w1_ref, bf1_ref, w2_ref, bf2_ref,
             erep_ref, ered_ref,
             out_ref, gat_ref):
    erep = erep_ref[...]
    ered = ered_ref[...]

    den16 = acc0_ref[:, D:] + acc1_ref[:, D:]
    den = jnp.dot(den16, erep, preferred_element_type=jnp.float32) + wrep_ref[...]
    num = acc0_ref[:, :D] + acc1_ref[:, :D] + snum_ref[...]
    gat = num / den
    gat_ref[...] = gat

    q = q_ref[...]
    k2 = jnp.dot(gat, wk_ref[...], preferred_element_type=jnp.float32) + bk_ref[...]
    v2 = jnp.dot(gat, wv_ref[...], preferred_element_type=jnp.float32) + bv_ref[...]
    s2 = jnp.dot(q * k2, ered, preferred_element_type=jnp.float32)
    a1 = 1.0 / (1.0 + jnp.exp(s2 - s1_ref[...]))
    a1r = jnp.dot(a1, erep, preferred_element_type=jnp.float32)
    o = v2 + a1r * (v1_ref[...] - v2)
    ca = jnp.dot(o, wo_ref[...], preferred_element_type=jnp.float32) + bo_ref[...]
    x = _ln(x1_ref[...] + ca, g2_ref[...], b2_ref[...])

    hid = jnp.maximum(
        jnp.dot(x, w1_ref[...], preferred_element_type=jnp.float32)
        + bf1_ref[...], 0.0)
    ff = jnp.dot(hid, w2_ref[...], preferred_element_type=jnp.float32) + bf2_ref[...]
    out_ref[...] = _ln(x + ff, g3_ref[...], b3_ref[...])


def _row_specs(shapes):
    return [pl.BlockSpec(s, lambda i: (i, 0)) for s in shapes]


def _full_specs(arrs):
    return [pl.BlockSpec(a.shape, lambda i: (0, 0)) for a in arrs]


def _finalize(acc0, acc1, pre, weights):
    nb = 25
    B = N_DST // nb
    in_specs = (_row_specs([(B, AW), (B, AW), (B, D), (B, D), (B, D),
                            (B, 16), (B, D), (B, D)])
                + _full_specs(weights))
    return pl.pallas_call(
        _c2_body,
        grid=(nb,),
        in_specs=in_specs,
        out_specs=_row_specs([(B, D), (B, D)]),
        out_shape=[
            jax.ShapeDtypeStruct((N_DST, D), jnp.float32),
            jax.ShapeDtypeStruct((N_DST, D), jnp.float32),
        ],
    )(acc0, acc1, *pre, *weights)


def kernel(feat_src, feat_dst, params, edge_index):
    rows = jnp.arange(D)
    colmask = (rows[:, None] // HD) == jnp.arange(16)[None, :]
    almat = jnp.where(colmask, params["attn_l"].reshape(-1)[:, None], 0.0)
    armat = jnp.where(colmask, params["attn_r"].reshape(-1)[:, None], 0.0)
    erep = colmask.astype(jnp.float32).T
    ered = colmask.astype(jnp.float32) * 0.25

    wv = rows // 32
    rem = rows % 32
    half = rem // 16
    ii = rem % 16
    pcol = 32 * wv + 2 * ii + half
    permat = (pcol[:, None] == rows[None, :]).astype(jnp.float32)

    c32 = jnp.arange(32)
    al32 = jnp.where(
        ((rows[:, None] // HD) == (c32[None, :] // 2)) & (c32[None, :] % 2 == 0),
        params["attn_l"].reshape(-1)[:, None], 0.0)

    pad = E_PAD - E
    src_p = jnp.concatenate(
        [edge_index[0].astype(jnp.int32), jnp.zeros((pad,), jnp.int32)])
    dst_p = jnp.concatenate(
        [edge_index[1].astype(jnp.int32),
         jnp.full((pad,), N_DST, jnp.int32)])

    sa_p, ca_p = params["sa"], params["ca"]

    def r1(v):
        return v.reshape(1, -1)

    w_c1 = [
        sa_p["Wv"], r1(sa_p["bv"]), sa_p["Wo"], r1(sa_p["bo"]),
        ca_p["Wq"], r1(ca_p["bq"]), ca_p["Wk"], r1(ca_p["bk"]),
        ca_p["Wv"], r1(ca_p["bv"]),
        r1(params["ln1_g"]), r1(params["ln1_b"]),
        erep, ered,
    ]
    hbel, er_dst, *pre = _gat_pre(feat_src, feat_dst, params["W_gat"], al32,
                                  permat, almat, armat, w_c1)

    acc0, acc1 = _sc_edge(hbel, er_dst, src_p, dst_p)

    w_c2 = [
        ca_p["Wk"], r1(ca_p["bk"]), ca_p["Wv"], r1(ca_p["bv"]),
        ca_p["Wo"], r1(ca_p["bo"]),
        r1(params["ln2_g"]), r1(params["ln2_b"]),
        r1(params["ln3_g"]), r1(params["ln3_b"]),
        params["W1"], r1(params["b1"]), params["W2"], r1(params["b2"]),
        erep, ered,
    ]
    out, gat = _finalize(acc0, acc1, pre, w_c2)
    return out, gat
